# baseline (speedup 1.0000x reference)
import functools

import jax
import jax.numpy as jnp
from jax import lax
from jax.experimental import pallas as pl
from jax.experimental.pallas import tpu as pltpu
from jax.experimental.pallas import tpu_sc as plsc

_NUM_CORES = 2
_NUM_SUBCORES = 16
_LANES = 16
_NUM_WORKERS = _NUM_CORES * _NUM_SUBCORES
_CHUNKS = 8


def _make_lookup(n_total, table_size):
    per_w = n_total // _NUM_WORKERS
    chunk = per_w // _CHUNKS
    mesh = plsc.VectorSubcoreMesh(core_axis_name="c", subcore_axis_name="s")

    @functools.partial(
        pl.kernel,
        mesh=mesh,
        out_type=jax.ShapeDtypeStruct((n_total,), jnp.int32),
        compiler_params=pltpu.CompilerParams(needs_layout_passes=False),
        scratch_types=[
            pltpu.VMEM((table_size,), jnp.int32),
            pltpu.VMEM((per_w,), jnp.int32),
            pltpu.VMEM((per_w,), jnp.int32),
            pltpu.SemaphoreType.DMA((_CHUNKS,)),
            pltpu.SemaphoreType.DMA((_CHUNKS,)),
        ],
    )
    def lookup(idx_hbm, table_hbm, out_hbm, tbl_v, idx_v, out_v, in_sems, out_sems):
        wid = lax.axis_index("s") * _NUM_CORES + lax.axis_index("c")
        base = wid * per_w
        in_handles = [
            pltpu.async_copy(
                idx_hbm.at[pl.ds(base + c * chunk, chunk)],
                idx_v.at[pl.ds(c * chunk, chunk)],
                in_sems.at[c],
            )
            for c in range(_CHUNKS)
        ]
        pltpu.sync_copy(table_hbm, tbl_v)
        head = tbl_v[pl.ds(0, _LANES)]
        lane = lax.iota(jnp.int32, _LANES)
        tbl_v[pl.ds(0, _LANES)] = jnp.where(lane == 0, 0, head)

        out_handles = []
        for c in range(_CHUNKS):
            in_handles[c].wait()

            @plsc.parallel_loop(c * chunk, (c + 1) * chunk, _LANES, unroll=16)
            def body(off):
                ix = idx_v[pl.ds(off, _LANES)]
                out_v[pl.ds(off, _LANES)] = plsc.load_gather(tbl_v, [ix])

            out_handles.append(
                pltpu.async_copy(
                    out_v.at[pl.ds(c * chunk, chunk)],
                    out_hbm.at[pl.ds(base + c * chunk, chunk)],
                    out_sems.at[c],
                )
            )
        for h in out_handles:
            h.wait()

    return lookup


def kernel(inputs, table):
    flat = inputs.reshape(-1).astype(jnp.int32)
    out = _make_lookup(flat.shape[0], table.shape[0])(flat, table)
    return out.reshape(inputs.shape)


# speedup vs baseline: 1.0205x; 1.0205x over previous
import functools

import jax
import jax.numpy as jnp
from jax import lax
from jax.experimental import pallas as pl
from jax.experimental.pallas import tpu as pltpu
from jax.experimental.pallas import tpu_sc as plsc

_NUM_CORES = 2
_NUM_SUBCORES = 16
_LANES = 16
_NUM_WORKERS = _NUM_CORES * _NUM_SUBCORES
_CHUNKS = 4


def _make_lookup(n_total, table_size):
    per_w = n_total // _NUM_WORKERS
    chunk = per_w // _CHUNKS
    mesh = plsc.VectorSubcoreMesh(core_axis_name="c", subcore_axis_name="s")

    @functools.partial(
        pl.kernel,
        mesh=mesh,
        out_type=jax.ShapeDtypeStruct((n_total,), jnp.int32),
        compiler_params=pltpu.CompilerParams(needs_layout_passes=False),
        scratch_types=[
            pltpu.VMEM((table_size,), jnp.int32),
            pltpu.VMEM((per_w,), jnp.int32),
            pltpu.VMEM((per_w,), jnp.int32),
            pltpu.SemaphoreType.DMA((_CHUNKS,)),
            pltpu.SemaphoreType.DMA((_CHUNKS,)),
        ],
    )
    def lookup(idx_hbm, table_hbm, out_hbm, tbl_v, idx_v, out_v, in_sems, out_sems):
        wid = lax.axis_index("s") * _NUM_CORES + lax.axis_index("c")
        base = wid * per_w
        in_handles = [
            pltpu.async_copy(
                idx_hbm.at[pl.ds(base + c * chunk, chunk)],
                idx_v.at[pl.ds(c * chunk, chunk)],
                in_sems.at[c],
            )
            for c in range(_CHUNKS)
        ]
        pltpu.sync_copy(table_hbm, tbl_v)
        head = tbl_v[pl.ds(0, _LANES)]
        lane = lax.iota(jnp.int32, _LANES)
        tbl_v[pl.ds(0, _LANES)] = jnp.where(lane == 0, 0, head)

        out_handles = []
        for c in range(_CHUNKS):
            in_handles[c].wait()

            @plsc.parallel_loop(c * chunk, (c + 1) * chunk, _LANES, unroll=8)
            def body(off):
                ix = idx_v[pl.ds(off, _LANES)]
                out_v[pl.ds(off, _LANES)] = plsc.load_gather(tbl_v, [ix])

            out_handles.append(
                pltpu.async_copy(
                    out_v.at[pl.ds(c * chunk, chunk)],
                    out_hbm.at[pl.ds(base + c * chunk, chunk)],
                    out_sems.at[c],
                )
            )
        for h in out_handles:
            h.wait()

    return lookup


def kernel(inputs, table):
    flat = inputs.reshape(-1).astype(jnp.int32)
    out = _make_lookup(flat.shape[0], table.shape[0])(flat, table)
    return out.reshape(inputs.shape)
